# baseline (speedup 1.0000x reference)
import functools

import jax
import jax.numpy as jnp
from jax import lax
from jax.experimental import pallas as pl
from jax.experimental.pallas import tpu as pltpu
from jax.experimental.pallas import tpu_sc as plsc

VOCAB = 128000
D_MODEL = 4096
NTOK = 16384

_info = plsc.get_sparse_core_info()
NC, NS = _info.num_cores, _info.num_subcores
NW = NC * NS
TPW = NTOK // NW
K = 8
NCHUNKS = TPW // K
NBUF = 3
NTRIPLES = (NCHUNKS - 1) // NBUF


@functools.partial(
    pl.kernel,
    mesh=plsc.VectorSubcoreMesh(core_axis_name="c", subcore_axis_name="s"),
    out_type=jax.ShapeDtypeStruct((NTOK, D_MODEL), jnp.float32),
    scratch_types=[
        pltpu.VMEM((TPW,), jnp.int32),
        pltpu.VMEM((K, D_MODEL), jnp.float32),
        pltpu.VMEM((K, D_MODEL), jnp.float32),
        pltpu.VMEM((K, D_MODEL), jnp.float32),
        pltpu.SemaphoreType.DMA,
        pltpu.SemaphoreType.DMA,
        pltpu.SemaphoreType.DMA,
        pltpu.SemaphoreType.DMA,
        pltpu.SemaphoreType.DMA,
        pltpu.SemaphoreType.DMA,
    ],
)
def _emb_lookup(
    ids_hbm, table_hbm, out_hbm, idx_v, buf0, buf1, buf2, gs0, gs1, gs2, ws0, ws1, ws2
):
    wid = lax.axis_index("s") * NC + lax.axis_index("c")
    base = wid * TPW
    pltpu.sync_copy(ids_hbm.at[pl.ds(base, TPW)], idx_v)
    bufs = (buf0, buf1, buf2)
    gs = (gs0, gs1, gs2)
    ws = (ws0, ws1, ws2)

    def gather(c, j):
        pltpu.async_copy(table_hbm.at[idx_v.at[pl.ds(c * K, K)]], bufs[j], gs[j])

    def wait_g(j):
        pltpu.make_async_copy(table_hbm.at[pl.ds(0, K)], bufs[j], gs[j]).wait()

    def write(c, j):
        pltpu.async_copy(bufs[j], out_hbm.at[pl.ds(base + c * K, K)], ws[j])

    def wait_w(j):
        pltpu.make_async_copy(table_hbm.at[pl.ds(0, K)], bufs[j], ws[j]).wait()

    def step(c, j, g):
        @pl.when(g > 0)
        def _():
            wait_w(j)

        gather(c, j)
        wait_g(j)
        write(c, j)

    def round_body(g, carry):
        for j in range(NBUF):
            step(NBUF * g + j, j, g)
        return carry

    lax.fori_loop(0, NTRIPLES, round_body, 0)
    step(NCHUNKS - 1, 0, NTRIPLES)
    for j in range(NBUF):
        wait_w(j)


def kernel(input_ids, embed_weight):
    ids_flat = input_ids.reshape(NTOK).astype(jnp.int32)
    out = _emb_lookup(ids_flat, embed_weight)
    return out.reshape(input_ids.shape[0], input_ids.shape[1], D_MODEL)


# speedup vs baseline: 1.0100x; 1.0100x over previous
import functools

import jax
import jax.numpy as jnp
from jax import lax
from jax.experimental import pallas as pl
from jax.experimental.pallas import tpu as pltpu
from jax.experimental.pallas import tpu_sc as plsc

VOCAB = 128000
D_MODEL = 4096
NTOK = 16384

_info = plsc.get_sparse_core_info()
NC, NS = _info.num_cores, _info.num_subcores
NW = NC * NS
TPW = NTOK // NW
K = 8
NCHUNKS = TPW // K
NBUF = 3
NTRIPLES = (NCHUNKS - 1) // NBUF


@functools.partial(
    pl.kernel,
    mesh=plsc.VectorSubcoreMesh(core_axis_name="c", subcore_axis_name="s"),
    out_type=jax.ShapeDtypeStruct((NTOK, D_MODEL), jnp.float32),
    scratch_types=[
        pltpu.VMEM((TPW,), jnp.int32),
        pltpu.VMEM((K, D_MODEL), jnp.float32),
        pltpu.VMEM((K, D_MODEL), jnp.float32),
        pltpu.VMEM((K, D_MODEL), jnp.float32),
        pltpu.SemaphoreType.DMA,
        pltpu.SemaphoreType.DMA,
        pltpu.SemaphoreType.DMA,
    ],
)
def _emb_lookup(ids_hbm, table_hbm, out_hbm, idx_v, buf0, buf1, buf2, sem0, sem1, sem2):
    wid = lax.axis_index("s") * NC + lax.axis_index("c")
    base = wid * TPW
    pltpu.sync_copy(ids_hbm.at[pl.ds(base, TPW)], idx_v)
    bufs = (buf0, buf1, buf2)
    sems = (sem0, sem1, sem2)

    def gather(c, j):
        pltpu.async_copy(table_hbm.at[idx_v.at[pl.ds(c * K, K)]], bufs[j], sems[j])

    def wait(j):
        pltpu.make_async_copy(table_hbm.at[pl.ds(0, K)], bufs[j], sems[j]).wait()

    def write_out(c, j):
        pltpu.sync_copy(bufs[j], out_hbm.at[pl.ds(base + c * K, K)])

    for j in range(NBUF):
        gather(j, j)

    def ring_body(g, carry):
        for j in range(NBUF):
            c = NBUF * g + j
            wait(j)
            write_out(c, j)

            @pl.when(c + NBUF < NCHUNKS)
            def _():
                gather(c + NBUF, j)

        return carry

    lax.fori_loop(0, NTRIPLES, ring_body, 0)
    wait(0)
    write_out(NCHUNKS - 1, 0)


def kernel(input_ids, embed_weight):
    ids_flat = input_ids.reshape(NTOK).astype(jnp.int32)
    out = _emb_lookup(ids_flat, embed_weight)
    return out.reshape(input_ids.shape[0], input_ids.shape[1], D_MODEL)
